# Initial kernel scaffold; baseline (speedup 1.0000x reference)
import jax
import jax.numpy as jnp
from jax.experimental import pallas as pl


def kernel(x, h, edge_index, edge_attr, W_enc, b_enc, W1, b1, W2, b2, Wa, ba, Wn1, bn1, Wn2, bn2, W_dec, b_dec, Wg1, bg1, Wg2, bg2, Wg3, bg3):
    raise NotImplementedError("write your pallas kernel here")


# speedup vs baseline: 1.8179x; 1.8179x over previous
import functools

import jax
import jax.numpy as jnp
from jax import lax
from jax.experimental import pallas as pl
from jax.experimental.pallas import tpu as pltpu
from jax.experimental.pallas import tpu_sc as plsc

N = 50000
E = 800000
IN_D = 16
D = 64
MSG = 64
ED = 4
L = 4
G = 128

NC = 2
NS = 16
NW = NC * NS

SLAB = 128
NSLAB = E // SLAB
GRP = 8
NGRP = NSLAB // GRP
GPW = -(-NGRP // NW)
SPW = -(-NGRP // NS)

HALF = N // NC
QN = (12000, 13000)
QSTART = (0, 12000)
QROWS = (12032, 13056)


def _sc_mesh():
    return plsc.VectorSubcoreMesh(
        core_axis_name="c", subcore_axis_name="s", num_cores=NC, num_subcores=NS)


def _make_gather2(d):
    ch = 1

    def body(tA, iA, tB, iB, outA, outB, idxA, idxB, bufA, bufB, semA, semB):
        c = lax.axis_index("c")
        s = lax.axis_index("s")
        w = s * NC + c

        def do_slabs(slab0, nsl):
            pltpu.sync_copy(iA.at[pl.ds(slab0, nsl)], idxA.at[pl.ds(0, nsl)])
            pltpu.sync_copy(iB.at[pl.ds(slab0, nsl)], idxB.at[pl.ds(0, nsl)])
            for c0 in range(0, nsl, ch):
                n = min(ch, nsl - c0)
                cps = []
                for j in range(n):
                    cps.append(pltpu.async_copy(
                        tA.at[idxA.at[c0 + j]],
                        bufA.at[pl.ds(j * SLAB, SLAB)], semA))
                    cps.append(pltpu.async_copy(
                        tB.at[idxB.at[c0 + j]],
                        bufB.at[pl.ds(j * SLAB, SLAB)], semB))
                for cp in cps:
                    cp.wait()
                e0 = (slab0 + c0) * SLAB
                nres = n * SLAB
                pltpu.sync_copy(bufA.at[pl.ds(0, nres)],
                                outA.at[pl.ds(e0, nres)])
                pltpu.sync_copy(bufB.at[pl.ds(0, nres)],
                                outB.at[pl.ds(e0, nres)])

        def do_macro(mi, _):
            g = mi * NW + w
            @pl.when(g < NGRP)
            def _():
                do_slabs(g * GRP, GRP)
            return 0

        lax.fori_loop(0, GPW, do_macro, 0)

        @pl.when(w == 0)
        def _tail():
            do_slabs(NGRP * GRP, NSLAB - NGRP * GRP)

    return pl.kernel(
        body,
        out_type=(jax.ShapeDtypeStruct((E, d), jnp.float32),
                  jax.ShapeDtypeStruct((E, d), jnp.float32)),
        mesh=_sc_mesh(),
        scratch_types=(
            pltpu.VMEM((GRP, SLAB), jnp.int32),
            pltpu.VMEM((GRP, SLAB), jnp.int32),
            pltpu.VMEM((ch * SLAB, d), jnp.float32),
            pltpu.VMEM((ch * SLAB, d), jnp.float32),
            pltpu.SemaphoreType.DMA,
            pltpu.SemaphoreType.DMA,
        ),
    )


def _make_scatter(call):
    qn = QN[call]
    qstart = QSTART[call]
    qrows = QROWS[call]
    rpt = qrows // NS
    nfull, rem = divmod(rpt, SLAB)

    def body(m, im, agg, idxm, idx2, mbuf, shared):
        c = lax.axis_index("c")
        s = lax.axis_index("s")
        base = c * HALF + qstart

        def zrow(r, _):
            for g in range(8):
                mbuf[r, pl.ds(g * 16, 16)] = jnp.zeros((16,), jnp.float32)
            return 0
        lax.fori_loop(0, SLAB, zrow, 0)
        for t in range(nfull):
            pltpu.sync_copy(mbuf.at[pl.ds(0, SLAB)],
                            shared.at[pl.ds(s * rpt + t * SLAB, SLAB)])
        if rem:
            pltpu.sync_copy(mbuf.at[pl.ds(0, rem)],
                            shared.at[pl.ds(s * rpt + nfull * SLAB, rem)])
        plsc.subcore_barrier()

        def do_slab_group(slab0, nsl):
            pltpu.sync_copy(im.at[pl.ds(slab0, nsl)], idxm.at[pl.ds(0, nsl)])
            for j in range(nsl):
                pltpu.sync_copy(m.at[pl.ds((slab0 + j) * SLAB, SLAB)], mbuf)
                for g in range(8):
                    v = idxm[j, pl.ds(g * 16, 16)]
                    li = v - base
                    ok = (li >= 0) & (li < qn)
                    trash = qn + jnp.bitwise_and(v, 15)
                    idx2[pl.ds(g * 16, 16)] = jnp.where(ok, li, trash)
                pltpu.sync_copy(mbuf, shared.at[idx2], add=True)

        def do_macro(mi, _):
            g = mi * NS + s
            @pl.when(g < NGRP)
            def _():
                do_slab_group(g * GRP, GRP)
            return 0
        lax.fori_loop(0, SPW, do_macro, 0)

        @pl.when(s == 0)
        def _tail():
            do_slab_group(NGRP * GRP, NSLAB - NGRP * GRP)

        plsc.subcore_barrier()
        for t in range(nfull):
            r0 = s * rpt + t * SLAB
            pltpu.sync_copy(shared.at[pl.ds(r0, SLAB)],
                            agg.at[c, pl.ds(r0, SLAB)])
        if rem:
            r0 = s * rpt + nfull * SLAB
            pltpu.sync_copy(shared.at[pl.ds(r0, rem)],
                            agg.at[c, pl.ds(r0, rem)])

    return pl.kernel(
        body,
        out_type=jax.ShapeDtypeStruct((NC, qrows, 128), jnp.float32),
        mesh=_sc_mesh(),
        scratch_types=(
            pltpu.VMEM((GRP, SLAB), jnp.int32),
            pltpu.VMEM((SLAB,), jnp.int32),
            pltpu.VMEM((SLAB, 128), jnp.float32),
            pltpu.VMEM_SHARED((qrows, 128), jnp.float32),
        ),
    )


_BN = 1000
_BE = 2000


def _dot(a, b):
    return jnp.dot(a, b, preferred_element_type=jnp.float32)


def _tc_encode(h, W_enc, b_enc):
    def body(h_ref, w_ref, b_ref, o_ref):
        o_ref[...] = _dot(h_ref[...], w_ref[...]) + b_ref[...]
    return pl.pallas_call(
        body,
        grid=(N // _BN,),
        in_specs=[pl.BlockSpec((_BN, IN_D), lambda i: (i, 0)),
                  pl.BlockSpec((IN_D, D), lambda i: (0, 0)),
                  pl.BlockSpec((1, D), lambda i: (0, 0))],
        out_specs=pl.BlockSpec((_BN, D), lambda i: (i, 0)),
        out_shape=jax.ShapeDtypeStruct((N, D), jnp.float32),
    )(h, W_enc, b_enc)


def _tc_pq(h, W1ab):
    def body(h_ref, w_ref, t_ref):
        t_ref[...] = _dot(h_ref[...], w_ref[...])
    return pl.pallas_call(
        body,
        grid=(N // _BN,),
        in_specs=[pl.BlockSpec((_BN, D), lambda i: (i, 0)),
                  pl.BlockSpec((D, 2 * MSG), lambda i: (0, 0))],
        out_specs=pl.BlockSpec((_BN, 2 * MSG), lambda i: (i, 0)),
        out_shape=jax.ShapeDtypeStruct((N, 2 * MSG), jnp.float32),
    )(h, W1ab)


def _tc_radial(xr, xc):
    def body(r_ref, c_ref, o_ref):
        d = r_ref[...] - c_ref[...]
        o_ref[...] = jnp.sum(d * d, axis=1, keepdims=True)
    return pl.pallas_call(
        body,
        grid=(E // 4000,),
        in_specs=[pl.BlockSpec((4000, 128), lambda i: (i, 0)),
                  pl.BlockSpec((4000, 128), lambda i: (i, 0))],
        out_specs=pl.BlockSpec((4000, 1), lambda i: (i, 0)),
        out_shape=jax.ShapeDtypeStruct((E, 1), jnp.float32),
    )(xr, xc)


def _silu(x):
    return x * jax.nn.sigmoid(x)


def _tc_edge(eR, eC, ea5, W5, b1, W2, b2, Wa, ba):
    def body(r_ref, c_ref, e_ref, w5_ref, b1_ref, w2_ref, b2_ref,
             wa_ref, ba_ref, o_ref):
        u = (r_ref[:, :MSG] + c_ref[:, MSG:]
             + _dot(e_ref[...], w5_ref[...]) + b1_ref[...])
        u = _silu(u)
        mm = _silu(_dot(u, w2_ref[...]) + b2_ref[...])
        att = jax.nn.sigmoid(_dot(mm, wa_ref[...]) + ba_ref[...])
        o_ref[:, :MSG] = mm * att
        o_ref[:, MSG:] = jnp.zeros_like(mm)
    return pl.pallas_call(
        body,
        grid=(E // _BE,),
        in_specs=[pl.BlockSpec((_BE, 2 * MSG), lambda i: (i, 0)),
                  pl.BlockSpec((_BE, 2 * MSG), lambda i: (i, 0)),
                  pl.BlockSpec((_BE, ED + 1), lambda i: (i, 0)),
                  pl.BlockSpec((ED + 1, MSG), lambda i: (0, 0)),
                  pl.BlockSpec((1, MSG), lambda i: (0, 0)),
                  pl.BlockSpec((MSG, MSG), lambda i: (0, 0)),
                  pl.BlockSpec((1, MSG), lambda i: (0, 0)),
                  pl.BlockSpec((MSG, 1), lambda i: (0, 0)),
                  pl.BlockSpec((1, 1), lambda i: (0, 0))],
        out_specs=pl.BlockSpec((_BE, 2 * MSG), lambda i: (i, 0)),
        out_shape=jax.ShapeDtypeStruct((E, 2 * MSG), jnp.float32),
    )(eR, eC, ea5, W5, b1, W2, b2, Wa, ba)


def _tc_node(h, agg0, agg1, Wn1a, Wn1b, bn1, Wn2, bn2):
    nb = N // _BN
    nb0 = QN[0] // _BN

    def body(h_ref, a0_ref, a1_ref, wa_ref, wb_ref, b1_ref, w2_ref, b2_ref,
             o_ref):
        i = pl.program_id(0)
        j = i % 25
        hv = h_ref[...]
        av = jnp.where(j < nb0, a0_ref[0][:, :MSG], a1_ref[0][:, :MSG])
        t = _silu(_dot(hv, wa_ref[...]) + _dot(av, wb_ref[...]) + b1_ref[...])
        o_ref[...] = _dot(t, w2_ref[...]) + b2_ref[...] + hv

    return pl.pallas_call(
        body,
        grid=(nb,),
        in_specs=[pl.BlockSpec((_BN, D), lambda i: (i, 0)),
                  pl.BlockSpec((1, _BN, 128),
                               lambda i: (i // 25, jnp.minimum(i % 25, 11), 0)),
                  pl.BlockSpec((1, _BN, 128),
                               lambda i: (i // 25,
                                          jnp.clip((i % 25) - 12, 0, 12), 0)),
                  pl.BlockSpec((D, D), lambda i: (0, 0)),
                  pl.BlockSpec((MSG, D), lambda i: (0, 0)),
                  pl.BlockSpec((1, D), lambda i: (0, 0)),
                  pl.BlockSpec((D, D), lambda i: (0, 0)),
                  pl.BlockSpec((1, D), lambda i: (0, 0))],
        out_specs=pl.BlockSpec((_BN, D), lambda i: (i, 0)),
        out_shape=jax.ShapeDtypeStruct((N, D), jnp.float32),
    )(h, agg0, agg1, Wn1a, Wn1b, bn1, Wn2, bn2)


def _tc_decode(h, W_dec, b_dec, Wg1, bg1, Wg2, bg2, Wg3, bg3):
    nb = N // _BN
    def body(h_ref, wd_ref, bd_ref, wg1_ref, bg1_ref, wg2_ref, bg2_ref,
             wg3_ref, bg3_ref, o_ref, acc):
        i = pl.program_id(0)

        @pl.when(i == 0)
        def _():
            acc[...] = jnp.zeros_like(acc)

        hd = _dot(h_ref[...], wd_ref[...]) + bd_ref[...]
        acc[0:1, :] += lax.dot_general(
            hd, wg1_ref[...], (((0,), (0,)), ((), ())),
            preferred_element_type=jnp.float32)

        @pl.when(i == nb - 1)
        def _():
            g = jnp.maximum(acc[0:1, :] + bg1_ref[...], 0.0)
            g = jnp.maximum(_dot(g, wg2_ref[...]) + bg2_ref[...], 0.0)
            o_ref[...] = _dot(g, wg3_ref[...]) + bg3_ref[...]

    return pl.pallas_call(
        body,
        grid=(nb,),
        in_specs=[pl.BlockSpec((_BN, D), lambda i: (i, 0)),
                  pl.BlockSpec((D, 1), lambda i: (0, 0)),
                  pl.BlockSpec((1, 1), lambda i: (0, 0)),
                  pl.BlockSpec((_BN, G), lambda i: (i, 0)),
                  pl.BlockSpec((1, G), lambda i: (0, 0)),
                  pl.BlockSpec((G, G), lambda i: (0, 0)),
                  pl.BlockSpec((1, G), lambda i: (0, 0)),
                  pl.BlockSpec((G, 1), lambda i: (0, 0)),
                  pl.BlockSpec((1, 1), lambda i: (0, 0))],
        out_specs=pl.BlockSpec((1, 1), lambda i: (0, 0)),
        out_shape=jax.ShapeDtypeStruct((1, 1), jnp.float32),
        scratch_shapes=[pltpu.VMEM((8, G), jnp.float32)],
    )(h, W_dec, b_dec, Wg1, bg1, Wg2, bg2, Wg3, bg3)


_gather2 = _make_gather2(128)
_scatter0 = _make_scatter(0)
_scatter1 = _make_scatter(1)


def kernel(x, h, edge_index, edge_attr, W_enc, b_enc, W1, b1, W2, b2, Wa, ba,
           Wn1, bn1, Wn2, bn2, W_dec, b_dec, Wg1, bg1, Wg2, bg2, Wg3, bg3):
    row = edge_index[0].reshape(NSLAB, SLAB)
    col = edge_index[1].reshape(NSLAB, SLAB)

    xpad = jnp.pad(x, ((0, 0), (0, 128 - 3)))
    xr, xc = _gather2(xpad, row, xpad, col)
    radial = _tc_radial(xr, xc)
    ea5 = jnp.concatenate([edge_attr, radial], axis=1)

    hh = _tc_encode(h, W_enc, b_enc.reshape(1, D))

    for l in range(L):
        W5 = jnp.concatenate([W1[l, 2 * D + 1:], W1[l, 2 * D:2 * D + 1]], axis=0)
        T = _tc_pq(hh, W1[l, :2 * D])
        eR, eC = _gather2(T, row, T, col)
        m = _tc_edge(eR, eC, ea5, W5, b1[l].reshape(1, MSG),
                     W2[l], b2[l].reshape(1, MSG), Wa[l], ba[l].reshape(1, 1))
        agg0 = _scatter0(m, row)
        agg1 = _scatter1(m, row)
        hh = _tc_node(hh, agg0, agg1, Wn1[l, :D], Wn1[l, D:],
                      bn1[l].reshape(1, D), Wn2[l], bn2[l].reshape(1, D))

    return _tc_decode(hh, W_dec, b_dec.reshape(1, 1),
                      Wg1, bg1.reshape(1, G), Wg2, bg2.reshape(1, G),
                      Wg3, bg3.reshape(1, 1))
